# baseline (speedup 1.0000x reference)
import functools

import jax
import jax.numpy as jnp
from jax import lax
from jax.experimental import pallas as pl
from jax.experimental.pallas import tpu as pltpu
from jax.experimental.pallas import tpu_sc as plsc

B = 16384
NC = 26
NN = 13
V = 100000
D = 32
NF = NC + NN

NCORES = 2
NSUB = 16
NW = NCORES * NSUB
ROWS_W = B // NW
GU = ROWS_W * NC // 128
NU = ROWS_W * NN // 128
CHUNK = 128
NCHUNK = ROWS_W // CHUNK


def _sc_body(xn, gidx, orow, nrow, tab, w, b, out,
             idx_v, oidx_v, nidx_v, xbuf, wv, bv, gbuf, numbuf,
             sem_g, sem_s, sem_n):
    wid = lax.axis_index("s") * NCORES + lax.axis_index("c")

    pltpu.sync_copy(gidx.at[wid], idx_v)
    pltpu.sync_copy(orow.at[wid], oidx_v)
    pltpu.sync_copy(nrow.at[wid], nidx_v)
    pltpu.sync_copy(xn.at[wid], xbuf)
    pltpu.sync_copy(w, wv)
    pltpu.sync_copy(b, bv)

    def cat_group(g, _):
        u0 = g * 4
        hs = []
        for k in range(4):
            hs.append(pltpu.async_copy(tab.at[idx_v.at[u0 + k]], gbuf.at[k],
                                       sem_g))
        for k in range(4):
            hs[k].wait()
        ss = []
        for k in range(4):
            ss.append(pltpu.async_copy(gbuf.at[k], out.at[oidx_v.at[u0 + k]],
                                       sem_s))
        for k in range(4):
            ss[k].wait()
        return 0

    lax.fori_loop(0, GU // 4, cat_group, 0)

    for c in range(NCHUNK):
        def num_body(bi, _):
            for j in range(NN):
                sel = jnp.full((16,), (c * CHUNK + bi) * NN + j,
                               dtype=jnp.int32)
                vx = plsc.load_gather(xbuf, [sel])
                for h in range(2):
                    wvv = wv[pl.ds(j * D + h * 16, 16)]
                    bvv = bv[pl.ds(j * D + h * 16, 16)]
                    numbuf[bi * NN + j, pl.ds(h * 16, 16)] = vx * wvv + bvv
            return 0

        lax.fori_loop(0, CHUNK, num_body, 0)
        ns = []
        for r in range(NN):
            ns.append(pltpu.async_copy(
                numbuf.at[pl.ds(r * 128, 128)],
                out.at[nidx_v.at[c * NN + r]], sem_n))
        for r in range(NN):
            ns[r].wait()


@jax.jit
def _tokenize(xn, gidx, orow, nrow, tab, w, b):
    return pl.kernel(
        _sc_body,
        out_type=jax.ShapeDtypeStruct((B * NF, D), jnp.float32),
        mesh=plsc.VectorSubcoreMesh(core_axis_name="c", subcore_axis_name="s",
                                    num_cores=NCORES, num_subcores=NSUB),
        compiler_params=pltpu.CompilerParams(needs_layout_passes=False,
                                             use_tc_tiling_on_sc=False),
        scratch_types=[
            pltpu.VMEM((GU, 128), jnp.int32),
            pltpu.VMEM((GU, 128), jnp.int32),
            pltpu.VMEM((NU, 128), jnp.int32),
            pltpu.VMEM((ROWS_W * NN,), jnp.float32),
            pltpu.VMEM((NN * D,), jnp.float32),
            pltpu.VMEM((NN * D,), jnp.float32),
            pltpu.VMEM((4, 128, D), jnp.float32),
            pltpu.VMEM((CHUNK * NN, D), jnp.float32),
            pltpu.SemaphoreType.DMA,
            pltpu.SemaphoreType.DMA,
            pltpu.SemaphoreType.DMA,
        ],
    )(xn, gidx, orow, nrow, tab, w, b)


def kernel(x_num, x_cat, cat_tables, num_W, num_b):
    gidx = (x_cat.astype(jnp.int32)
            + (jnp.arange(NC, dtype=jnp.int32) * V)[None, :])
    gidx = gidx.reshape(NW, GU, 128)
    bb = jnp.arange(B, dtype=jnp.int32)[:, None] * NF
    orow = (bb + jnp.arange(NC, dtype=jnp.int32)[None, :])
    orow = orow.reshape(NW, GU, 128)
    nrow = (bb + NC + jnp.arange(NN, dtype=jnp.int32)[None, :])
    nrow = nrow.reshape(NW, NU, 128)
    tab = cat_tables.reshape(NC * V, D)
    xn = x_num.reshape(NW, ROWS_W * NN)
    out = _tokenize(xn, gidx, orow, nrow, tab,
                    num_W.reshape(NN * D), num_b.reshape(NN * D))
    return out.reshape(B, NF, D)


# speedup vs baseline: 1.6413x; 1.6413x over previous
import jax
import jax.numpy as jnp
from jax import lax
from jax.experimental import pallas as pl
from jax.experimental.pallas import tpu as pltpu
from jax.experimental.pallas import tpu_sc as plsc

B = 16384
NC = 26
NN = 13
V = 100000
D = 32
NF = NC + NN

NCORES = 2
NSUB = 16
HB = 8192


def _sc_body(tab2, xc1, xn1, rown, w1, b1, out,
             staged, idx_v, obuf, rown_v, wv, bvv, sem_g, sem_o):
    c = lax.axis_index("c")
    s = lax.axis_index("s")
    d = c * NSUB + s
    pltpu.sync_copy(w1, wv)
    pltpu.sync_copy(b1, bvv)
    pltpu.sync_copy(rown.at[d], rown_v)
    prev = [None]
    for f in range(NF):
        if f < NC:
            pltpu.async_copy(tab2.at[rown_v.at[f]], staged, sem_g).wait()
            wsp = bsp = None
        else:
            sel = jnp.full((16,), d * NN + (f - NC), dtype=jnp.int32)
            wsp = plsc.load_gather(wv, [sel])
            bsp = plsc.load_gather(bvv, [sel])
        zero16 = jnp.zeros((16,), dtype=jnp.int32)
        for h in range(2):
            if f < NC:
                pltpu.sync_copy(xc1.at[pl.ds(f * B + h * HB, HB)], idx_v)
            else:
                pltpu.sync_copy(xn1.at[pl.ds((f - NC) * B + h * HB, HB)],
                                idx_v)
            if prev[0] is not None and h == 0:
                prev[0].wait()
            if f < NC:
                def gbody(t, _, h=h):
                    vi = idx_v[pl.ds(t * 16, 16)]
                    obuf[0, pl.ds(h * HB + t * 16, 16)] = (
                        plsc.load_gather(staged, [zero16, vi]))
                    return 0
                lax.fori_loop(0, HB // 16, gbody, 0)
            else:
                def nbody(t, _, h=h, wsp=wsp, bsp=bsp):
                    vx = plsc.bitcast(idx_v[pl.ds(t * 16, 16)], jnp.float32)
                    obuf[0, pl.ds(h * HB + t * 16, 16)] = vx * wsp + bsp
                    return 0
                lax.fori_loop(0, HB // 16, nbody, 0)
        prev[0] = pltpu.async_copy(obuf, out.at[rown_v.at[f]], sem_o)
    prev[0].wait()


@jax.jit
def _tokenize(tab2, xc1, xn1, rown, w1, b1):
    return pl.kernel(
        _sc_body,
        out_type=jax.ShapeDtypeStruct((NF * D, B), jnp.float32),
        mesh=plsc.VectorSubcoreMesh(core_axis_name="c", subcore_axis_name="s",
                                    num_cores=NCORES, num_subcores=NSUB),
        compiler_params=pltpu.CompilerParams(needs_layout_passes=False,
                                             use_tc_tiling_on_sc=False),
        scratch_types=[
            pltpu.VMEM((1, V), jnp.float32),
            pltpu.VMEM((HB,), jnp.int32),
            pltpu.VMEM((1, B), jnp.float32),
            pltpu.VMEM((40, 1), jnp.int32),
            pltpu.VMEM((NN * D,), jnp.float32),
            pltpu.VMEM((NN * D,), jnp.float32),
            pltpu.SemaphoreType.DMA,
            pltpu.SemaphoreType.DMA,
        ],
    )(tab2, xc1, xn1, rown, w1, b1)


def kernel(x_num, x_cat, cat_tables, num_W, num_b):
    tab2 = cat_tables.transpose(0, 2, 1).reshape(NC * D, V)
    xc1 = x_cat.astype(jnp.int32).T.reshape(NC * B)
    xn1 = lax.bitcast_convert_type(x_num, jnp.int32).T.reshape(NN * B)
    w1 = num_W.T.reshape(NN * D)
    b1 = num_b.T.reshape(NN * D)
    rown = (jnp.arange(D, dtype=jnp.int32)[:, None]
            + jnp.arange(NF + 1, dtype=jnp.int32)[None, :] * D)
    rown = rown.reshape(D, NF + 1, 1)
    outp = _tokenize(tab2, xc1, xn1, rown, w1, b1)
    return outp.reshape(NF, D, B).transpose(2, 0, 1)


# speedup vs baseline: 1.8450x; 1.1242x over previous
import jax
import jax.numpy as jnp
from jax import lax
from jax.experimental import pallas as pl
from jax.experimental.pallas import tpu as pltpu
from jax.experimental.pallas import tpu_sc as plsc

B = 16384
NC = 26
NN = 13
V = 100000
D = 32
NF = NC + NN

NCORES = 2
NSUB = 16
HB = 8192


def _sc_body(tab2, xc1, xn1, rown, w1, b1, out,
             staged, idx_v, obuf, rown_v, wv, bvv, sem_g, sem_o):
    c = lax.axis_index("c")
    s = lax.axis_index("s")
    d = c * NSUB + s
    pltpu.sync_copy(w1, wv)
    pltpu.sync_copy(b1, bvv)
    pltpu.sync_copy(rown.at[d], rown_v)
    zero16 = jnp.zeros((16,), dtype=jnp.int32)
    prev = [None]
    stage = [None]

    def fire_stage(f):
        stage[0] = pltpu.async_copy(tab2.at[rown_v.at[f]], staged, sem_g)

    def run_unit(f, wsp, bsp):
        for h in range(2):
            if f < NC:
                pltpu.sync_copy(xc1.at[pl.ds(f * B + h * HB, HB)], idx_v)
            else:
                pltpu.sync_copy(xn1.at[pl.ds((f - NC) * B + h * HB, HB)],
                                idx_v)
            if prev[0] is not None and h == 0:
                prev[0].wait()
            if f < NC:
                def gbody(t, _, h=h):
                    for k in range(4):
                        vi = idx_v[pl.ds(t * 64 + k * 16, 16)]
                        obuf[0, pl.ds(h * HB + t * 64 + k * 16, 16)] = (
                            plsc.load_gather(staged, [zero16, vi]))
                    return 0
                lax.fori_loop(0, HB // 64, gbody, 0)
            else:
                def nbody(t, _, h=h, wsp=wsp, bsp=bsp):
                    for k in range(4):
                        vx = plsc.bitcast(
                            idx_v[pl.ds(t * 64 + k * 16, 16)], jnp.float32)
                        obuf[0, pl.ds(h * HB + t * 64 + k * 16, 16)] = (
                            vx * wsp + bsp)
                    return 0
                lax.fori_loop(0, HB // 64, nbody, 0)
        prev[0] = pltpu.async_copy(obuf, out.at[rown_v.at[f]], sem_o)

    order = []
    ni = 0
    for k in range(NC):
        order.append(k)
        if k % 2 == 1 and ni < NN:
            order.append(NC + ni)
            ni += 1
    fire_stage(0)
    for f in order:
        if f < NC:
            stage[0].wait()
            nxt = f + 1 if f + 1 < NC else None
            run_unit(f, None, None)
            if nxt is not None:
                pass
        else:
            sel = jnp.full((16,), d * NN + (f - NC), dtype=jnp.int32)
            run_unit(f, plsc.load_gather(wv, [sel]),
                     plsc.load_gather(bvv, [sel]))
        if f < NC and f + 1 < NC:
            fire_stage(f + 1)
    prev[0].wait()


@jax.jit
def _tokenize(tab2, xc1, xn1, rown, w1, b1):
    return pl.kernel(
        _sc_body,
        out_type=jax.ShapeDtypeStruct((NF * D, B), jnp.float32),
        mesh=plsc.VectorSubcoreMesh(core_axis_name="c", subcore_axis_name="s",
                                    num_cores=NCORES, num_subcores=NSUB),
        compiler_params=pltpu.CompilerParams(needs_layout_passes=False,
                                             use_tc_tiling_on_sc=False),
        scratch_types=[
            pltpu.VMEM((1, V), jnp.float32),
            pltpu.VMEM((HB,), jnp.int32),
            pltpu.VMEM((1, B), jnp.float32),
            pltpu.VMEM((40, 1), jnp.int32),
            pltpu.VMEM((NN * D,), jnp.float32),
            pltpu.VMEM((NN * D,), jnp.float32),
            pltpu.SemaphoreType.DMA,
            pltpu.SemaphoreType.DMA,
        ],
    )(tab2, xc1, xn1, rown, w1, b1)


def kernel(x_num, x_cat, cat_tables, num_W, num_b):
    tab2 = cat_tables.transpose(0, 2, 1).reshape(NC * D, V)
    xc1 = x_cat.astype(jnp.int32).T.reshape(NC * B)
    xn1 = lax.bitcast_convert_type(x_num, jnp.int32).T.reshape(NN * B)
    w1 = num_W.T.reshape(NN * D)
    b1 = num_b.T.reshape(NN * D)
    rown = (jnp.arange(D, dtype=jnp.int32)[:, None]
            + jnp.arange(NF + 1, dtype=jnp.int32)[None, :] * D)
    rown = rown.reshape(D, NF + 1, 1)
    outp = _tokenize(tab2, xc1, xn1, rown, w1, b1)
    return outp.reshape(NF, D, B).transpose(2, 0, 1)


# speedup vs baseline: 3.5936x; 1.9477x over previous
import jax
import jax.numpy as jnp
from jax import lax
from jax.experimental import pallas as pl
from jax.experimental.pallas import tpu as pltpu
from jax.experimental.pallas import tpu_sc as plsc

B = 16384
NC = 26
NN = 13
V = 100000
D = 32
NF = NC + NN

NCORES = 2
NSUB = 16
HB = 8192
VA = 99968


def _sc_body(tab2, xc1, xn1, rown, tail1, w1, b1, out,
             staged, idx_v, obuf, rown_v, wv, bvv, sem_g, sem_o):
    c = lax.axis_index("c")
    s = lax.axis_index("s")
    d = c * NSUB + s
    pltpu.sync_copy(w1, wv)
    pltpu.sync_copy(b1, bvv)
    pltpu.sync_copy(rown.at[d], rown_v)
    zero16 = jnp.zeros((16,), dtype=jnp.int32)
    prev = [None]
    stage = [None]

    def fire_stage(f):
        stage[0] = pltpu.async_copy(
            tab2.at[rown_v.at[f], pl.ds(0, VA)], staged.at[:, pl.ds(0, VA)],
            sem_g)
        pltpu.sync_copy(tail1.at[pl.ds(d * (NC * D) + f * D, D)],
                        staged.at[0, pl.ds(VA, D)])

    def run_unit(f, wsp, bsp):
        for h in range(2):
            if f < NC:
                pltpu.sync_copy(xc1.at[pl.ds(f * B + h * HB, HB)], idx_v)
            else:
                pltpu.sync_copy(xn1.at[pl.ds((f - NC) * B + h * HB, HB)],
                                idx_v)
            if prev[0] is not None and h == 0:
                prev[0].wait()
            if f < NC:
                def gbody(t, _, h=h):
                    for k in range(4):
                        vi = idx_v[pl.ds(t * 64 + k * 16, 16)]
                        obuf[0, pl.ds(h * HB + t * 64 + k * 16, 16)] = (
                            plsc.load_gather(staged, [zero16, vi]))
                    return 0
                lax.fori_loop(0, HB // 64, gbody, 0)
            else:
                def nbody(t, _, h=h, wsp=wsp, bsp=bsp):
                    for k in range(4):
                        vx = plsc.bitcast(
                            idx_v[pl.ds(t * 64 + k * 16, 16)], jnp.float32)
                        obuf[0, pl.ds(h * HB + t * 64 + k * 16, 16)] = (
                            vx * wsp + bsp)
                    return 0
                lax.fori_loop(0, HB // 64, nbody, 0)
        prev[0] = pltpu.async_copy(obuf, out.at[rown_v.at[f]], sem_o)

    order = []
    ni = 0
    for k in range(NC):
        order.append(k)
        if k % 2 == 1 and ni < NN:
            order.append(NC + ni)
            ni += 1
    fire_stage(0)
    for f in order:
        if f < NC:
            stage[0].wait()
            nxt = f + 1 if f + 1 < NC else None
            run_unit(f, None, None)
            if nxt is not None:
                pass
        else:
            sel = jnp.full((16,), d * NN + (f - NC), dtype=jnp.int32)
            run_unit(f, plsc.load_gather(wv, [sel]),
                     plsc.load_gather(bvv, [sel]))
        if f < NC and f + 1 < NC:
            fire_stage(f + 1)
    prev[0].wait()


@jax.jit
def _tokenize(tab2, xc1, xn1, rown, tail1, w1, b1):
    return pl.kernel(
        _sc_body,
        out_type=jax.ShapeDtypeStruct((NF * D, B), jnp.float32),
        mesh=plsc.VectorSubcoreMesh(core_axis_name="c", subcore_axis_name="s",
                                    num_cores=NCORES, num_subcores=NSUB),
        compiler_params=pltpu.CompilerParams(needs_layout_passes=False,
                                             use_tc_tiling_on_sc=True),
        scratch_types=[
            pltpu.VMEM((1, V), jnp.float32),
            pltpu.VMEM((HB,), jnp.int32),
            pltpu.VMEM((1, B), jnp.float32),
            pltpu.VMEM((40, 1), jnp.int32),
            pltpu.VMEM((NN * D,), jnp.float32),
            pltpu.VMEM((NN * D,), jnp.float32),
            pltpu.SemaphoreType.DMA,
            pltpu.SemaphoreType.DMA,
        ],
    )(tab2, xc1, xn1, rown, tail1, w1, b1)


def kernel(x_num, x_cat, cat_tables, num_W, num_b):
    tab2 = cat_tables.transpose(0, 2, 1).reshape(NC * D, V)
    xc1 = x_cat.astype(jnp.int32).T.reshape(NC * B)
    xn1 = lax.bitcast_convert_type(x_num, jnp.int32).T.reshape(NN * B)
    w1 = num_W.T.reshape(NN * D)
    b1 = num_b.T.reshape(NN * D)
    rown = (jnp.arange(D, dtype=jnp.int32)[:, None]
            + jnp.arange(NF + 1, dtype=jnp.int32)[None, :] * D)
    rown = rown.reshape(D, NF + 1, 1)
    tail1 = cat_tables[:, VA:, :].transpose(2, 0, 1).reshape(D * NC * D)
    outp = _tokenize(tab2, xc1, xn1, rown, tail1, w1, b1)
    return outp.reshape(NF, D, B).transpose(2, 0, 1)


# speedup vs baseline: 4.6125x; 1.2835x over previous
import jax
import jax.numpy as jnp
from jax import lax
from jax.experimental import pallas as pl
from jax.experimental.pallas import tpu as pltpu
from jax.experimental.pallas import tpu_sc as plsc

B = 16384
NC = 26
NN = 13
V = 100000
D = 32
NF = NC + NN

NCORES = 2
NSUB = 16
HB = 8192
VA = 99968


def _sc_body(tab2, xc1, xn1, rown, tail1, w1, b1, out,
             staged, idx_v, obuf0, obuf1, rown_v, wv, bvv, sem_g, sem_o):
    c = lax.axis_index("c")
    s = lax.axis_index("s")
    d = c * NSUB + s
    pltpu.sync_copy(w1, wv)
    pltpu.sync_copy(b1, bvv)
    pltpu.sync_copy(rown.at[d], rown_v)
    zero16 = jnp.zeros((16,), dtype=jnp.int32)
    stage = [None]
    prevh = [None, None]

    def fire_stage(f):
        stage[0] = pltpu.async_copy(
            tab2.at[rown_v.at[f], pl.ds(0, VA)], staged.at[:, pl.ds(0, VA)],
            sem_g)
        pltpu.sync_copy(tail1.at[pl.ds(d * (NC * D) + f * D, D)],
                        staged.at[0, pl.ds(VA, D)])

    def half_unit(f, h, buf):
        obuf = obuf0 if buf == 0 else obuf1
        if f < NC:
            pltpu.sync_copy(xc1.at[pl.ds(f * B + h * HB, HB)], idx_v)
            wsp = bsp = None
        else:
            pltpu.sync_copy(xn1.at[pl.ds((f - NC) * B + h * HB, HB)], idx_v)
            sel = jnp.full((16,), d * NN + (f - NC), dtype=jnp.int32)
            wsp = plsc.load_gather(wv, [sel])
            bsp = plsc.load_gather(bvv, [sel])
        if prevh[buf] is not None:
            prevh[buf].wait()
        if f < NC:
            def gbody(t, _):
                for k in range(4):
                    vi = idx_v[pl.ds(t * 64 + k * 16, 16)]
                    obuf[0, pl.ds(t * 64 + k * 16, 16)] = (
                        plsc.load_gather(staged, [zero16, vi]))
                return 0
            lax.fori_loop(0, HB // 64, gbody, 0)
        else:
            def nbody(t, _):
                for k in range(4):
                    vx = plsc.bitcast(
                        idx_v[pl.ds(t * 64 + k * 16, 16)], jnp.float32)
                    obuf[0, pl.ds(t * 64 + k * 16, 16)] = vx * wsp + bsp
                return 0
            lax.fori_loop(0, HB // 64, nbody, 0)
        prevh[buf] = pltpu.async_copy(
            obuf, out.at[rown_v.at[f], pl.ds(h * HB, HB)], sem_o)

    fillers = [(NC + j, h) for j in range(NN) for h in range(2)]
    fire_stage(0)
    nbuf = 0
    for f in range(NC):
        stage[0].wait()
        for h in range(2):
            half_unit(f, h, nbuf)
            nbuf ^= 1
        if f + 1 < NC:
            fire_stage(f + 1)
        nf, nh = fillers[f]
        half_unit(nf, nh, nbuf)
        nbuf ^= 1
    for buf in range(2):
        if prevh[buf] is not None:
            prevh[buf].wait()


@jax.jit
def _tokenize(tab2, xc1, xn1, rown, tail1, w1, b1):
    return pl.kernel(
        _sc_body,
        out_type=jax.ShapeDtypeStruct((NF * D, B), jnp.float32),
        mesh=plsc.VectorSubcoreMesh(core_axis_name="c", subcore_axis_name="s",
                                    num_cores=NCORES, num_subcores=NSUB),
        compiler_params=pltpu.CompilerParams(needs_layout_passes=False,
                                             use_tc_tiling_on_sc=True),
        scratch_types=[
            pltpu.VMEM((1, V), jnp.float32),
            pltpu.VMEM((HB,), jnp.int32),
            pltpu.VMEM((1, HB), jnp.float32),
            pltpu.VMEM((1, HB), jnp.float32),
            pltpu.VMEM((40, 1), jnp.int32),
            pltpu.VMEM((NN * D,), jnp.float32),
            pltpu.VMEM((NN * D,), jnp.float32),
            pltpu.SemaphoreType.DMA,
            pltpu.SemaphoreType.DMA,
        ],
    )(tab2, xc1, xn1, rown, tail1, w1, b1)


def kernel(x_num, x_cat, cat_tables, num_W, num_b):
    tab2 = cat_tables.transpose(0, 2, 1).reshape(NC * D, V)
    xc1 = x_cat.astype(jnp.int32).T.reshape(NC * B)
    xn1 = lax.bitcast_convert_type(x_num, jnp.int32).T.reshape(NN * B)
    w1 = num_W.T.reshape(NN * D)
    b1 = num_b.T.reshape(NN * D)
    rown = (jnp.arange(D, dtype=jnp.int32)[:, None]
            + jnp.arange(NF + 1, dtype=jnp.int32)[None, :] * D)
    rown = rown.reshape(D, NF + 1, 1)
    tail1 = cat_tables[:, VA:, :].transpose(2, 0, 1).reshape(D * NC * D)
    outp = _tokenize(tab2, xc1, xn1, rown, tail1, w1, b1)
    return outp.reshape(NF, D, B).transpose(2, 0, 1)
